# Initial kernel scaffold; baseline (speedup 1.0000x reference)
import jax
import jax.numpy as jnp
from jax.experimental import pallas as pl


def kernel(pi, mu, log_sig):
    raise NotImplementedError("write your pallas kernel here")


# speedup vs baseline: 1.1405x; 1.1405x over previous
import functools

import numpy as np
import jax
import jax.numpy as jnp
from jax import lax
from jax.experimental import pallas as pl
from jax.experimental.pallas import tpu as pltpu
from jax.experimental.pallas import tpu_sc as plsc

B = 128
NC = 100000
W = 2048
NBLK = (NC + W - 1) // W

_KN0, _KN1 = 1832780943, 270669613
_KC0, _KC1 = 64467757, 2916123636

_TINY = float(np.finfo(np.float32).tiny)
_LO = float(np.nextafter(np.float32(-1.0), np.float32(0.0)))
_SPAN = float(np.float32(1.0) - np.float32(_LO))
_SQRT2 = float(np.float32(np.sqrt(2.0)))
_IMAX = np.int32(2**31 - 1)


def _threefry_bits(p_u32, k0, k1):
    k2 = k0 ^ k1 ^ 0x1BD11BDA
    ks = (k0, k1, k2)
    rots = ((13, 15, 26, 6), (17, 29, 16, 24))
    x0 = jnp.full_like(p_u32, jnp.uint32(k0))
    x1 = p_u32 + jnp.uint32(k1)
    for g in range(5):
        for d in rots[g % 2]:
            x0 = x0 + x1
            x1 = (x1 << jnp.uint32(d)) | (x1 >> jnp.uint32(32 - d))
            x1 = x1 ^ x0
        x0 = x0 + jnp.uint32(ks[(g + 1) % 3])
        x1 = x1 + jnp.uint32((ks[(g + 2) % 3] + g + 1) & 0xFFFFFFFF)
    return x0 ^ x1


def _bits_to_unit(bits):
    fb = (bits >> jnp.uint32(9)) | jnp.uint32(0x3F800000)
    return lax.bitcast_convert_type(fb, jnp.float32) - jnp.float32(1.0)


def _select_kernel(pi_ref, flat_ref, eps_ref, rmax_ref, ridx_ref):
    j = pl.program_id(0)

    @pl.when(j == 0)
    def _init():
        rmax_ref[...] = jnp.full((B, 1), -jnp.inf, jnp.float32)
        ridx_ref[...] = jnp.zeros((B, 1), jnp.int32)

    col = lax.broadcasted_iota(jnp.int32, (B, W), 1) + j * W
    row = lax.broadcasted_iota(jnp.int32, (B, W), 0)
    p = (row * NC + col).astype(jnp.uint32)
    bits = _threefry_bits(p, _KC0, _KC1)
    fl = _bits_to_unit(bits)
    u = jnp.maximum(jnp.float32(_TINY), fl + jnp.float32(_TINY))
    g = -jnp.log(-jnp.log(u))
    score = jnp.log(pi_ref[...] + jnp.float32(1e-12)) + g
    score = jnp.where(col < NC, score, -jnp.inf)

    bmax = jnp.max(score, axis=1, keepdims=True)
    bidx = jnp.min(jnp.where(score == bmax, col, _IMAX), axis=1, keepdims=True)
    better = bmax > rmax_ref[...]
    ridx_ref[...] = jnp.where(better, bidx, ridx_ref[...])
    rmax_ref[...] = jnp.maximum(bmax, rmax_ref[...])

    @pl.when(j == NBLK - 1)
    def _finish():
        idx = ridx_ref[...]
        rowc = lax.broadcasted_iota(jnp.int32, (B, 1), 0)
        flat = rowc * NC + idx
        bitsn = _threefry_bits(flat.astype(jnp.uint32), _KN0, _KN1)
        fln = _bits_to_unit(bitsn)
        un = jnp.maximum(jnp.float32(_LO),
                         fln * jnp.float32(_SPAN) + jnp.float32(_LO))
        eps_ref[...] = jnp.float32(_SQRT2) * lax.erf_inv(un)
        flat_ref[...] = flat


_select = pl.pallas_call(
    _select_kernel,
    grid=(NBLK,),
    in_specs=[pl.BlockSpec((B, W), lambda j: (0, j))],
    out_specs=[pl.BlockSpec((B, 1), lambda j: (0, 0)),
               pl.BlockSpec((B, 1), lambda j: (0, 0))],
    out_shape=[jax.ShapeDtypeStruct((B, 1), jnp.int32),
               jax.ShapeDtypeStruct((B, 1), jnp.float32)],
    scratch_shapes=[pltpu.VMEM((B, 1), jnp.float32),
                    pltpu.VMEM((B, 1), jnp.int32)],
)

_CH = 16
_NW = B // _CH


def _gather_body(mu_hbm, ls_hbm, flat_hbm, eps_hbm, out_hbm,
                 flat_v, mu_v, ls_v, eps_v, out_v, sem_mu, sem_ls):
    wid = lax.axis_index("s") * 2 + lax.axis_index("c")

    @pl.when(wid < _NW)
    def _():
        base = wid * _CH
        pltpu.sync_copy(flat_hbm.at[pl.ds(base, _CH)], flat_v)
        pltpu.sync_copy(eps_hbm.at[pl.ds(base, _CH)], eps_v)
        c_mu = pltpu.async_copy(mu_hbm.at[flat_v], mu_v, sem_mu)
        c_ls = pltpu.async_copy(ls_hbm.at[flat_v], ls_v, sem_ls)
        c_mu.wait()
        c_ls.wait()
        sig = jnp.exp(jnp.clip(ls_v[...], jnp.float32(-40.0),
                               jnp.float32(40.0)))
        out_v[...] = mu_v[...] + sig * eps_v[...]
        pltpu.sync_copy(out_v, out_hbm.at[pl.ds(base, _CH)])


@functools.cache
def _make_gather():
    return functools.partial(
        pl.kernel,
        mesh=plsc.VectorSubcoreMesh(core_axis_name="c", subcore_axis_name="s"),
        out_type=jax.ShapeDtypeStruct((B,), jnp.float32),
        scratch_types=[
            pltpu.VMEM((_CH,), jnp.int32),
            pltpu.VMEM((_CH,), jnp.float32),
            pltpu.VMEM((_CH,), jnp.float32),
            pltpu.VMEM((_CH,), jnp.float32),
            pltpu.VMEM((_CH,), jnp.float32),
            pltpu.SemaphoreType.DMA,
            pltpu.SemaphoreType.DMA,
        ],
    )(_gather_body)


def kernel(pi, mu, log_sig):
    flat, eps = _select(pi)
    sample = _make_gather()(mu.reshape(B * NC), log_sig.reshape(B * NC),
                            flat.reshape(B), eps.reshape(B))
    return sample.reshape(B, 1)


# speedup vs baseline: 1.4260x; 1.2504x over previous
import functools

import numpy as np
import jax
import jax.numpy as jnp
from jax import lax
from jax.experimental import pallas as pl
from jax.experimental.pallas import tpu as pltpu
from jax.experimental.pallas import tpu_sc as plsc

B = 128
NC = 100000
W = 2048
NBLK = (NC + W - 1) // W

_KN0, _KN1 = 1832780943, 270669613
_KC0, _KC1 = 64467757, 2916123636

_TINY = float(np.finfo(np.float32).tiny)
_LO = float(np.nextafter(np.float32(-1.0), np.float32(0.0)))
_SPAN = float(np.float32(1.0) - np.float32(_LO))
_SQRT2 = float(np.float32(np.sqrt(2.0)))
_IMAX = np.int32(2**31 - 1)


def _threefry_bits(p_u32, k0, k1):
    k2 = k0 ^ k1 ^ 0x1BD11BDA
    ks = (k0, k1, k2)
    rots = ((13, 15, 26, 6), (17, 29, 16, 24))
    x0 = jnp.full_like(p_u32, jnp.uint32(k0))
    x1 = p_u32 + jnp.uint32(k1)
    for g in range(5):
        for d in rots[g % 2]:
            x0 = x0 + x1
            x1 = (x1 << jnp.uint32(d)) | (x1 >> jnp.uint32(32 - d))
            x1 = x1 ^ x0
        x0 = x0 + jnp.uint32(ks[(g + 1) % 3])
        x1 = x1 + jnp.uint32((ks[(g + 2) % 3] + g + 1) & 0xFFFFFFFF)
    return x0 ^ x1


def _bits_to_unit(bits):
    fb = (bits >> jnp.uint32(9)) | jnp.uint32(0x3F800000)
    return lax.bitcast_convert_type(fb, jnp.float32) - jnp.float32(1.0)


def _select_kernel(pi_ref, flat_ref, eps_ref, rmax_ref, ridx_ref):
    j = pl.program_id(0)

    @pl.when(j == 0)
    def _init():
        rmax_ref[...] = jnp.full((B, 1), -jnp.inf, jnp.float32)
        ridx_ref[...] = jnp.zeros((B, 1), jnp.int32)

    col = lax.broadcasted_iota(jnp.int32, (B, W), 1) + j * W
    row = lax.broadcasted_iota(jnp.int32, (B, W), 0)
    p = (row * NC + col).astype(jnp.uint32)
    bits = _threefry_bits(p, _KC0, _KC1)
    fl = _bits_to_unit(bits)
    u = jnp.maximum(jnp.float32(_TINY), fl + jnp.float32(_TINY))
    g = -jnp.log(-jnp.log(u))
    score = jnp.log(pi_ref[...] + jnp.float32(1e-12)) + g
    score = jnp.where(col < NC, score, -jnp.inf)

    bmax = jnp.max(score, axis=1, keepdims=True)
    bidx = jnp.min(jnp.where(score == bmax, col, _IMAX), axis=1, keepdims=True)
    better = bmax > rmax_ref[...]
    ridx_ref[...] = jnp.where(better, bidx, ridx_ref[...])
    rmax_ref[...] = jnp.maximum(bmax, rmax_ref[...])

    @pl.when(j == NBLK - 1)
    def _finish():
        idx = ridx_ref[...]
        rowc = lax.broadcasted_iota(jnp.int32, (B, 1), 0)
        flat = rowc * NC + idx
        bitsn = _threefry_bits(flat.astype(jnp.uint32), _KN0, _KN1)
        fln = _bits_to_unit(bitsn)
        un = jnp.maximum(jnp.float32(_LO),
                         fln * jnp.float32(_SPAN) + jnp.float32(_LO))
        eps_ref[...] = jnp.float32(_SQRT2) * lax.erf_inv(un)
        flat_ref[...] = idx


_select = pl.pallas_call(
    _select_kernel,
    grid=(NBLK,),
    in_specs=[pl.BlockSpec((B, W), lambda j: (0, j))],
    out_specs=[pl.BlockSpec((B, 1), lambda j: (0, 0)),
               pl.BlockSpec((B, 1), lambda j: (0, 0))],
    out_shape=[jax.ShapeDtypeStruct((B, 1), jnp.int32),
               jax.ShapeDtypeStruct((B, 1), jnp.float32)],
    scratch_shapes=[pltpu.VMEM((B, 1), jnp.float32),
                    pltpu.VMEM((B, 1), jnp.int32)],
)

_GW = 128


def _gather_kernel(col_s, mu_ref, ls_ref, col_ref, eps_ref, out_ref,
                   bmu, bls, sem_mu, sem_ls):
    for i in range(B):
        c = pl.multiple_of(col_s[i] & ~(_GW - 1), _GW)
        r0 = (i // 8) * 8
        pltpu.make_async_copy(mu_ref.at[pl.ds(r0, 8), pl.ds(c, _GW)],
                              bmu.at[i], sem_mu).start()
        pltpu.make_async_copy(ls_ref.at[pl.ds(r0, 8), pl.ds(c, _GW)],
                              bls.at[i], sem_ls).start()
    for i in range(B):
        pltpu.make_async_copy(mu_ref.at[pl.ds(0, 8), pl.ds(0, _GW)],
                              bmu.at[i], sem_mu).wait()
        pltpu.make_async_copy(ls_ref.at[pl.ds(0, 8), pl.ds(0, _GW)],
                              bls.at[i], sem_ls).wait()
    col3 = col_ref[...].reshape(B, 1, 1)
    imod = lax.broadcasted_iota(jnp.int32, (B, 8, _GW), 0) & 7
    rio = lax.broadcasted_iota(jnp.int32, (B, 8, _GW), 1)
    lio = lax.broadcasted_iota(jnp.int32, (B, 8, _GW), 2)
    sel = (rio == imod) & (lio == (col3 & (_GW - 1)))
    muv = jnp.sum(jnp.where(sel, bmu[...], 0.0), axis=(1, 2))
    lsv = jnp.sum(jnp.where(sel, bls[...], 0.0), axis=(1, 2))
    sig = jnp.exp(jnp.clip(lsv, jnp.float32(-40.0), jnp.float32(40.0)))
    out_ref[...] = (muv + sig * eps_ref[...].reshape(B)).reshape(B, 1)


_gather = pl.pallas_call(
    _gather_kernel,
    grid_spec=pltpu.PrefetchScalarGridSpec(
        num_scalar_prefetch=1,
        grid=(1,),
        in_specs=[
            pl.BlockSpec(memory_space=pl.ANY),
            pl.BlockSpec(memory_space=pl.ANY),
            pl.BlockSpec((B, 1), lambda j, s: (0, 0)),
            pl.BlockSpec((B, 1), lambda j, s: (0, 0)),
        ],
        out_specs=pl.BlockSpec((B, 1), lambda j, s: (0, 0)),
        scratch_shapes=[pltpu.VMEM((B, 8, _GW), jnp.float32),
                        pltpu.VMEM((B, 8, _GW), jnp.float32),
                        pltpu.SemaphoreType.DMA,
                        pltpu.SemaphoreType.DMA],
    ),
    out_shape=jax.ShapeDtypeStruct((B, 1), jnp.float32),
)


def kernel(pi, mu, log_sig):
    col, eps = _select(pi)
    return _gather(col.reshape(B), mu, log_sig, col, eps)


# speedup vs baseline: 1.4270x; 1.0007x over previous
import functools

import numpy as np
import jax
import jax.numpy as jnp
from jax import lax
from jax.experimental import pallas as pl
from jax.experimental.pallas import tpu as pltpu
from jax.experimental.pallas import tpu_sc as plsc

B = 128
NC = 100000
W = 2048
NBLK = (NC + W - 1) // W

_KN0, _KN1 = 1832780943, 270669613
_KC0, _KC1 = 64467757, 2916123636

_TINY = float(np.finfo(np.float32).tiny)
_LO = float(np.nextafter(np.float32(-1.0), np.float32(0.0)))
_SPAN = float(np.float32(1.0) - np.float32(_LO))
_SQRT2 = float(np.float32(np.sqrt(2.0)))
_IMAX = np.int32(2**31 - 1)


def _threefry_bits(p_u32, k0, k1):
    k2 = k0 ^ k1 ^ 0x1BD11BDA
    ks = (k0, k1, k2)
    rots = ((13, 15, 26, 6), (17, 29, 16, 24))
    x0 = jnp.full_like(p_u32, jnp.uint32(k0))
    x1 = p_u32 + jnp.uint32(k1)
    for g in range(5):
        for d in rots[g % 2]:
            x0 = x0 + x1
            x1 = (x1 << jnp.uint32(d)) | (x1 >> jnp.uint32(32 - d))
            x1 = x1 ^ x0
        x0 = x0 + jnp.uint32(ks[(g + 1) % 3])
        x1 = x1 + jnp.uint32((ks[(g + 2) % 3] + g + 1) & 0xFFFFFFFF)
    return x0 ^ x1


def _bits_to_unit(bits):
    fb = (bits >> jnp.uint32(9)) | jnp.uint32(0x3F800000)
    return lax.bitcast_convert_type(fb, jnp.float32) - jnp.float32(1.0)


def _select_kernel(pi_ref, flat_ref, eps_ref, rmax_ref, ridx_ref):
    j = pl.program_id(0)

    @pl.when(j == 0)
    def _init():
        rmax_ref[...] = jnp.full((B, 1), -jnp.inf, jnp.float32)
        ridx_ref[...] = jnp.zeros((B, 1), jnp.int32)

    col = lax.broadcasted_iota(jnp.int32, (B, W), 1) + j * W
    row = lax.broadcasted_iota(jnp.int32, (B, W), 0)
    p = (row * NC + col).astype(jnp.uint32)
    bits = _threefry_bits(p, _KC0, _KC1)
    fl = _bits_to_unit(bits)
    u = jnp.maximum(jnp.float32(_TINY), fl + jnp.float32(_TINY))
    g = -jnp.log(-jnp.log(u))
    score = jnp.log(pi_ref[...] + jnp.float32(1e-12)) + g
    score = jnp.where(col < NC, score, -jnp.inf)

    bmax = jnp.max(score, axis=1, keepdims=True)
    bidx = jnp.min(jnp.where(score == bmax, col, _IMAX), axis=1, keepdims=True)
    better = bmax > rmax_ref[...]
    ridx_ref[...] = jnp.where(better, bidx, ridx_ref[...])
    rmax_ref[...] = jnp.maximum(bmax, rmax_ref[...])

    @pl.when(j == NBLK - 1)
    def _finish():
        idx = ridx_ref[...]
        rowc = lax.broadcasted_iota(jnp.int32, (B, 1), 0)
        flat = rowc * NC + idx
        bitsn = _threefry_bits(flat.astype(jnp.uint32), _KN0, _KN1)
        fln = _bits_to_unit(bitsn)
        un = jnp.maximum(jnp.float32(_LO),
                         fln * jnp.float32(_SPAN) + jnp.float32(_LO))
        eps_ref[...] = jnp.float32(_SQRT2) * lax.erf_inv(un)
        flat_ref[...] = idx


_select = pl.pallas_call(
    _select_kernel,
    grid=(NBLK,),
    in_specs=[pl.BlockSpec((B, W), lambda j: (0, j))],
    out_specs=[pl.BlockSpec((B, 1), lambda j: (0, 0)),
               pl.BlockSpec((B, 1), lambda j: (0, 0))],
    out_shape=[jax.ShapeDtypeStruct((B, 1), jnp.int32),
               jax.ShapeDtypeStruct((B, 1), jnp.float32)],
    scratch_shapes=[pltpu.VMEM((B, 1), jnp.float32),
                    pltpu.VMEM((B, 1), jnp.int32)],
)

_GW = 128


def _gather_kernel(col_s, mu_ref, ls_ref, col_ref, eps_ref, out_ref,
                   bmu, bls, sem_mu, sem_ls):
    for i in range(B):
        c = pl.multiple_of(col_s[i, 0] & ~(_GW - 1), _GW)
        r0 = (i // 8) * 8
        pltpu.make_async_copy(mu_ref.at[pl.ds(r0, 8), pl.ds(c, _GW)],
                              bmu.at[i], sem_mu).start()
        pltpu.make_async_copy(ls_ref.at[pl.ds(r0, 8), pl.ds(c, _GW)],
                              bls.at[i], sem_ls).start()
    for i in range(B):
        pltpu.make_async_copy(mu_ref.at[pl.ds(0, 8), pl.ds(0, _GW)],
                              bmu.at[i], sem_mu).wait()
        pltpu.make_async_copy(ls_ref.at[pl.ds(0, 8), pl.ds(0, _GW)],
                              bls.at[i], sem_ls).wait()
    col3 = col_ref[...].reshape(B, 1, 1)
    imod = lax.broadcasted_iota(jnp.int32, (B, 8, _GW), 0) & 7
    rio = lax.broadcasted_iota(jnp.int32, (B, 8, _GW), 1)
    lio = lax.broadcasted_iota(jnp.int32, (B, 8, _GW), 2)
    sel = (rio == imod) & (lio == (col3 & (_GW - 1)))
    muv = jnp.sum(jnp.where(sel, bmu[...], 0.0), axis=(1, 2))
    lsv = jnp.sum(jnp.where(sel, bls[...], 0.0), axis=(1, 2))
    sig = jnp.exp(jnp.clip(lsv, jnp.float32(-40.0), jnp.float32(40.0)))
    out_ref[...] = (muv + sig * eps_ref[...].reshape(B)).reshape(B, 1)


_gather = pl.pallas_call(
    _gather_kernel,
    grid_spec=pltpu.PrefetchScalarGridSpec(
        num_scalar_prefetch=1,
        grid=(1,),
        in_specs=[
            pl.BlockSpec(memory_space=pl.ANY),
            pl.BlockSpec(memory_space=pl.ANY),
            pl.BlockSpec((B, 1), lambda j, s: (0, 0)),
            pl.BlockSpec((B, 1), lambda j, s: (0, 0)),
        ],
        out_specs=pl.BlockSpec((B, 1), lambda j, s: (0, 0)),
        scratch_shapes=[pltpu.VMEM((B, 8, _GW), jnp.float32),
                        pltpu.VMEM((B, 8, _GW), jnp.float32),
                        pltpu.SemaphoreType.DMA,
                        pltpu.SemaphoreType.DMA],
    ),
    out_shape=jax.ShapeDtypeStruct((B, 1), jnp.float32),
)


def kernel(pi, mu, log_sig):
    col, eps = _select(pi)
    return _gather(col, mu, log_sig, col, eps)


# speedup vs baseline: 1.4437x; 1.0117x over previous
import numpy as np
import jax
import jax.numpy as jnp
from jax import lax
from jax.experimental import pallas as pl
from jax.experimental.pallas import tpu as pltpu

B = 128
NC = 100000
W = 2048
NBLK = (NC + W - 1) // W
_GW = 128

_KN0, _KN1 = 1832780943, 270669613
_KC0, _KC1 = 64467757, 2916123636

_TINY = float(np.finfo(np.float32).tiny)
_LO = float(np.nextafter(np.float32(-1.0), np.float32(0.0)))
_SPAN = float(np.float32(1.0) - np.float32(_LO))
_SQRT2 = float(np.float32(np.sqrt(2.0)))
_IMAX = np.int32(2**31 - 1)


def _threefry_bits(p_u32, k0, k1):
    k2 = k0 ^ k1 ^ 0x1BD11BDA
    ks = (k0, k1, k2)
    rots = ((13, 15, 26, 6), (17, 29, 16, 24))
    x0 = jnp.full_like(p_u32, jnp.uint32(k0))
    x1 = p_u32 + jnp.uint32(k1)
    for g in range(5):
        for d in rots[g % 2]:
            x0 = x0 + x1
            x1 = (x1 << jnp.uint32(d)) | (x1 >> jnp.uint32(32 - d))
            x1 = x1 ^ x0
        x0 = x0 + jnp.uint32(ks[(g + 1) % 3])
        x1 = x1 + jnp.uint32((ks[(g + 2) % 3] + g + 1) & 0xFFFFFFFF)
    return x0 ^ x1


def _bits_to_unit(bits):
    fb = (bits >> jnp.uint32(9)) | jnp.uint32(0x3F800000)
    return lax.bitcast_convert_type(fb, jnp.float32) - jnp.float32(1.0)


def _kernel(pi_ref, mu_ref, ls_ref, out_ref,
            rmax_ref, ridx_ref, bmu, bls, sem_mu, sem_ls):
    j = pl.program_id(0)

    @pl.when(j == 0)
    def _init():
        rmax_ref[...] = jnp.full((B, 1), -jnp.inf, jnp.float32)
        ridx_ref[...] = jnp.zeros((B, 1), jnp.int32)

    col = lax.broadcasted_iota(jnp.int32, (B, W), 1) + j * W
    row = lax.broadcasted_iota(jnp.int32, (B, W), 0)
    p = (row * NC + col).astype(jnp.uint32)
    bits = _threefry_bits(p, _KC0, _KC1)
    fl = _bits_to_unit(bits)
    u = jnp.maximum(jnp.float32(_TINY), fl + jnp.float32(_TINY))
    g = -jnp.log(-jnp.log(u))
    score = jnp.log(pi_ref[...] + jnp.float32(1e-12)) + g
    score = jnp.where(col < NC, score, -jnp.inf)

    bmax = jnp.max(score, axis=1, keepdims=True)
    bidx = jnp.min(jnp.where(score == bmax, col, _IMAX), axis=1, keepdims=True)
    better = bmax > rmax_ref[...]
    ridx_ref[...] = jnp.where(better, bidx, ridx_ref[...])
    rmax_ref[...] = jnp.maximum(bmax, rmax_ref[...])

    @pl.when(j == NBLK - 1)
    def _finish():
        for i in range(B):
            c = pl.multiple_of(ridx_ref[i, 0] & ~(_GW - 1), _GW)
            r0 = (i // 8) * 8
            pltpu.make_async_copy(mu_ref.at[pl.ds(r0, 8), pl.ds(c, _GW)],
                                  bmu.at[i], sem_mu).start()
            pltpu.make_async_copy(ls_ref.at[pl.ds(r0, 8), pl.ds(c, _GW)],
                                  bls.at[i], sem_ls).start()

        idx = ridx_ref[...]
        rowc = lax.broadcasted_iota(jnp.int32, (B, 1), 0)
        flat = rowc * NC + idx
        bitsn = _threefry_bits(flat.astype(jnp.uint32), _KN0, _KN1)
        fln = _bits_to_unit(bitsn)
        un = jnp.maximum(jnp.float32(_LO),
                         fln * jnp.float32(_SPAN) + jnp.float32(_LO))
        eps = jnp.float32(_SQRT2) * lax.erf_inv(un)

        for i in range(B):
            pltpu.make_async_copy(mu_ref.at[pl.ds(0, 8), pl.ds(0, _GW)],
                                  bmu.at[i], sem_mu).wait()
            pltpu.make_async_copy(ls_ref.at[pl.ds(0, 8), pl.ds(0, _GW)],
                                  bls.at[i], sem_ls).wait()

        col3 = idx.reshape(B, 1, 1)
        imod = lax.broadcasted_iota(jnp.int32, (B, 8, _GW), 0) & 7
        rio = lax.broadcasted_iota(jnp.int32, (B, 8, _GW), 1)
        lio = lax.broadcasted_iota(jnp.int32, (B, 8, _GW), 2)
        sel = (rio == imod) & (lio == (col3 & (_GW - 1)))
        muv = jnp.sum(jnp.where(sel, bmu[...], 0.0), axis=(1, 2))
        lsv = jnp.sum(jnp.where(sel, bls[...], 0.0), axis=(1, 2))
        sig = jnp.exp(jnp.clip(lsv, jnp.float32(-40.0), jnp.float32(40.0)))
        out_ref[...] = (muv + sig * eps.reshape(B)).reshape(B, 1)


_fused = pl.pallas_call(
    _kernel,
    grid=(NBLK,),
    in_specs=[pl.BlockSpec((B, W), lambda j: (0, j)),
              pl.BlockSpec(memory_space=pl.ANY),
              pl.BlockSpec(memory_space=pl.ANY)],
    out_specs=pl.BlockSpec((B, 1), lambda j: (0, 0)),
    out_shape=jax.ShapeDtypeStruct((B, 1), jnp.float32),
    scratch_shapes=[pltpu.VMEM((B, 1), jnp.float32),
                    pltpu.VMEM((B, 1), jnp.int32),
                    pltpu.VMEM((B, 8, _GW), jnp.float32),
                    pltpu.VMEM((B, 8, _GW), jnp.float32),
                    pltpu.SemaphoreType.DMA,
                    pltpu.SemaphoreType.DMA],
)


def kernel(pi, mu, log_sig):
    return _fused(pi, mu, log_sig)


# speedup vs baseline: 1.4520x; 1.0058x over previous
import numpy as np
import jax
import jax.numpy as jnp
from jax import lax
from jax.experimental import pallas as pl
from jax.experimental.pallas import tpu as pltpu

B = 128
NC = 100000
W = 5000
NBLK = NC // W

_KN0, _KN1 = 1832780943, 270669613
_KC0, _KC1 = 64467757, 2916123636

_TINY = float(np.finfo(np.float32).tiny)
_LO = float(np.nextafter(np.float32(-1.0), np.float32(0.0)))
_SPAN = float(np.float32(1.0) - np.float32(_LO))
_SQRT2 = float(np.float32(np.sqrt(2.0)))
_IMAX = np.int32(2**31 - 1)


def _threefry_bits(p_u32, k0, k1):
    k2 = k0 ^ k1 ^ 0x1BD11BDA
    ks = (k0, k1, k2)
    rots = ((13, 15, 26, 6), (17, 29, 16, 24))
    x0 = jnp.full_like(p_u32, jnp.uint32(k0))
    x1 = p_u32 + jnp.uint32(k1)
    for g in range(5):
        for d in rots[g % 2]:
            x0 = x0 + x1
            x1 = (x1 << jnp.uint32(d)) | (x1 >> jnp.uint32(32 - d))
            x1 = x1 ^ x0
        x0 = x0 + jnp.uint32(ks[(g + 1) % 3])
        x1 = x1 + jnp.uint32((ks[(g + 2) % 3] + g + 1) & 0xFFFFFFFF)
    return x0 ^ x1


def _bits_to_unit(bits):
    fb = (bits >> jnp.uint32(9)) | jnp.uint32(0x3F800000)
    return lax.bitcast_convert_type(fb, jnp.float32) - jnp.float32(1.0)


def _kernel(pit_ref, mut_ref, lst_ref, out_ref,
            rmax_ref, ridx_ref, bmu, bls, sem_mu, sem_ls):
    j = pl.program_id(0)

    @pl.when(j == 0)
    def _init():
        rmax_ref[...] = jnp.full((1, B), -jnp.inf, jnp.float32)
        ridx_ref[...] = jnp.zeros((1, B), jnp.int32)

    ri = lax.broadcasted_iota(jnp.int32, (W, B), 0) + j * W
    bi = lax.broadcasted_iota(jnp.int32, (W, B), 1)
    p = (bi * NC + ri).astype(jnp.uint32)
    bits = _threefry_bits(p, _KC0, _KC1)
    fl = _bits_to_unit(bits)
    u = fl + jnp.float32(_TINY)
    g = -jnp.log(-jnp.log(u))
    score = jnp.log(pit_ref[...] + jnp.float32(1e-12)) + g

    bmax = jnp.max(score, axis=0, keepdims=True)
    bidx = jnp.min(jnp.where(score == bmax, ri, _IMAX), axis=0, keepdims=True)
    better = bmax > rmax_ref[...]
    ridx_ref[...] = jnp.where(better, bidx, ridx_ref[...])
    rmax_ref[...] = jnp.maximum(bmax, rmax_ref[...])

    @pl.when(j == NBLK - 1)
    def _finish():
        for i in range(B):
            r0 = pl.multiple_of(ridx_ref[0, i] & ~7, 8)
            pltpu.make_async_copy(mut_ref.at[pl.ds(r0, 8)],
                                  bmu.at[i], sem_mu).start()
            pltpu.make_async_copy(lst_ref.at[pl.ds(r0, 8)],
                                  bls.at[i], sem_ls).start()

        idx = ridx_ref[...]
        bc = lax.broadcasted_iota(jnp.int32, (1, B), 1)
        flat = bc * NC + idx
        bitsn = _threefry_bits(flat.astype(jnp.uint32), _KN0, _KN1)
        fln = _bits_to_unit(bitsn)
        un = jnp.maximum(jnp.float32(_LO),
                         fln * jnp.float32(_SPAN) + jnp.float32(_LO))
        eps = jnp.float32(_SQRT2) * lax.erf_inv(un)

        for i in range(B):
            pltpu.make_async_copy(mut_ref.at[pl.ds(0, 8)],
                                  bmu.at[i], sem_mu).wait()
            pltpu.make_async_copy(lst_ref.at[pl.ds(0, 8)],
                                  bls.at[i], sem_ls).wait()

        idx3 = idx.reshape(1, 1, B)
        sio = lax.broadcasted_iota(jnp.int32, (B, 8, B), 0)
        rio = lax.broadcasted_iota(jnp.int32, (B, 8, B), 1)
        lio = lax.broadcasted_iota(jnp.int32, (B, 8, B), 2)
        sel = (sio == lio) & (rio == (idx3 & 7))
        muv = jnp.sum(jnp.where(sel, bmu[...], 0.0), axis=(0, 1))
        lsv = jnp.sum(jnp.where(sel, bls[...], 0.0), axis=(0, 1))
        sig = jnp.exp(jnp.clip(lsv, jnp.float32(-40.0), jnp.float32(40.0)))
        out_ref[...] = (muv + sig * eps.reshape(B)).reshape(1, B)


_fused = pl.pallas_call(
    _kernel,
    grid=(NBLK,),
    in_specs=[pl.BlockSpec((W, B), lambda j: (j, 0)),
              pl.BlockSpec(memory_space=pl.ANY),
              pl.BlockSpec(memory_space=pl.ANY)],
    out_specs=pl.BlockSpec((1, B), lambda j: (0, 0)),
    out_shape=jax.ShapeDtypeStruct((1, B), jnp.float32),
    scratch_shapes=[pltpu.VMEM((1, B), jnp.float32),
                    pltpu.VMEM((1, B), jnp.int32),
                    pltpu.VMEM((B, 8, B), jnp.float32),
                    pltpu.VMEM((B, 8, B), jnp.float32),
                    pltpu.SemaphoreType.DMA,
                    pltpu.SemaphoreType.DMA],
)


def kernel(pi, mu, log_sig):
    return _fused(pi.T, mu.T, log_sig.T).reshape(B, 1)


# speedup vs baseline: 1.4620x; 1.0069x over previous
import numpy as np
import jax
import jax.numpy as jnp
from jax import lax
from jax.experimental import pallas as pl
from jax.experimental.pallas import tpu as pltpu

B = 128
NC = 100000
W = 2000
NBLK = NC // W

_KN0, _KN1 = 1832780943, 270669613
_KC0, _KC1 = 64467757, 2916123636

_TINY = float(np.finfo(np.float32).tiny)
_LO = float(np.nextafter(np.float32(-1.0), np.float32(0.0)))
_SPAN = float(np.float32(1.0) - np.float32(_LO))
_SQRT2 = float(np.float32(np.sqrt(2.0)))
_IMAX = np.int32(2**31 - 1)


def _threefry_bits(p_u32, k0, k1):
    k2 = k0 ^ k1 ^ 0x1BD11BDA
    ks = (k0, k1, k2)
    rots = ((13, 15, 26, 6), (17, 29, 16, 24))
    x0 = jnp.full_like(p_u32, jnp.uint32(k0))
    x1 = p_u32 + jnp.uint32(k1)
    for g in range(5):
        for d in rots[g % 2]:
            x0 = x0 + x1
            x1 = (x1 << jnp.uint32(d)) | (x1 >> jnp.uint32(32 - d))
            x1 = x1 ^ x0
        x0 = x0 + jnp.uint32(ks[(g + 1) % 3])
        x1 = x1 + jnp.uint32((ks[(g + 2) % 3] + g + 1) & 0xFFFFFFFF)
    return x0 ^ x1


def _bits_to_unit(bits):
    fb = (bits >> jnp.uint32(9)) | jnp.uint32(0x3F800000)
    return lax.bitcast_convert_type(fb, jnp.float32) - jnp.float32(1.0)


def _kernel(pit_ref, mut_ref, lst_ref, out_ref,
            rmax_ref, ridx_ref, bmu, bls, sem_mu, sem_ls):
    j = pl.program_id(0)

    @pl.when(j == 0)
    def _init():
        rmax_ref[...] = jnp.full((1, B), -jnp.inf, jnp.float32)
        ridx_ref[...] = jnp.zeros((1, B), jnp.int32)

    ri = lax.broadcasted_iota(jnp.int32, (W, B), 0) + j * W
    bi = lax.broadcasted_iota(jnp.int32, (W, B), 1)
    p = (bi * NC + ri).astype(jnp.uint32)
    bits = _threefry_bits(p, _KC0, _KC1)
    fl = _bits_to_unit(bits)
    u = fl + jnp.float32(_TINY)
    g = -jnp.log(-jnp.log(u))
    score = jnp.log(pit_ref[...] + jnp.float32(1e-12)) + g

    bmax = jnp.max(score, axis=0, keepdims=True)
    bidx = jnp.min(jnp.where(score == bmax, ri, _IMAX), axis=0, keepdims=True)
    better = bmax > rmax_ref[...]
    ridx_ref[...] = jnp.where(better, bidx, ridx_ref[...])
    rmax_ref[...] = jnp.maximum(bmax, rmax_ref[...])

    @pl.when(j == NBLK - 1)
    def _finish():
        for i in range(B):
            r0 = pl.multiple_of(ridx_ref[0, i] & ~7, 8)
            pltpu.make_async_copy(mut_ref.at[pl.ds(r0, 8)],
                                  bmu.at[i], sem_mu).start()
            pltpu.make_async_copy(lst_ref.at[pl.ds(r0, 8)],
                                  bls.at[i], sem_ls).start()

        idx = ridx_ref[...]
        bc = lax.broadcasted_iota(jnp.int32, (1, B), 1)
        flat = bc * NC + idx
        bitsn = _threefry_bits(flat.astype(jnp.uint32), _KN0, _KN1)
        fln = _bits_to_unit(bitsn)
        un = jnp.maximum(jnp.float32(_LO),
                         fln * jnp.float32(_SPAN) + jnp.float32(_LO))
        eps = jnp.float32(_SQRT2) * lax.erf_inv(un)

        for i in range(B):
            pltpu.make_async_copy(mut_ref.at[pl.ds(0, 8)],
                                  bmu.at[i], sem_mu).wait()
            pltpu.make_async_copy(lst_ref.at[pl.ds(0, 8)],
                                  bls.at[i], sem_ls).wait()

        idx3 = idx.reshape(1, 1, B)
        sio = lax.broadcasted_iota(jnp.int32, (B, 8, B), 0)
        rio = lax.broadcasted_iota(jnp.int32, (B, 8, B), 1)
        lio = lax.broadcasted_iota(jnp.int32, (B, 8, B), 2)
        sel = (sio == lio) & (rio == (idx3 & 7))
        muv = jnp.sum(jnp.where(sel, bmu[...], 0.0), axis=(0, 1))
        lsv = jnp.sum(jnp.where(sel, bls[...], 0.0), axis=(0, 1))
        sig = jnp.exp(jnp.clip(lsv, jnp.float32(-40.0), jnp.float32(40.0)))
        out_ref[...] = (muv + sig * eps.reshape(B)).reshape(1, B)


_fused = pl.pallas_call(
    _kernel,
    grid=(NBLK,),
    in_specs=[pl.BlockSpec((W, B), lambda j: (j, 0)),
              pl.BlockSpec(memory_space=pl.ANY),
              pl.BlockSpec(memory_space=pl.ANY)],
    out_specs=pl.BlockSpec((1, B), lambda j: (0, 0)),
    out_shape=jax.ShapeDtypeStruct((1, B), jnp.float32),
    scratch_shapes=[pltpu.VMEM((1, B), jnp.float32),
                    pltpu.VMEM((1, B), jnp.int32),
                    pltpu.VMEM((B, 8, B), jnp.float32),
                    pltpu.VMEM((B, 8, B), jnp.float32),
                    pltpu.SemaphoreType.DMA,
                    pltpu.SemaphoreType.DMA],
)


def kernel(pi, mu, log_sig):
    return _fused(pi.T, mu.T, log_sig.T).reshape(B, 1)


# speedup vs baseline: 1.8955x; 1.2965x over previous
import numpy as np
import jax
import jax.numpy as jnp
from jax import lax
from jax.experimental import pallas as pl
from jax.experimental.pallas import tpu as pltpu

B = 128
NC = 100000
W = 2048
NBLK = (NC + W - 1) // W

_KN0, _KN1 = 1832780943, 270669613
_KC0, _KC1 = 64467757, 2916123636

_TINY = float(np.finfo(np.float32).tiny)
_LO = float(np.nextafter(np.float32(-1.0), np.float32(0.0)))
_SPAN = float(np.float32(1.0) - np.float32(_LO))
_SQRT2 = float(np.float32(np.sqrt(2.0)))
_IMAX = np.int32(2**31 - 1)


def _threefry_bits(p_u32, k0, k1):
    k2 = k0 ^ k1 ^ 0x1BD11BDA
    ks = (k0, k1, k2)
    rots = ((13, 15, 26, 6), (17, 29, 16, 24))
    x0 = jnp.full_like(p_u32, jnp.uint32(k0))
    x1 = p_u32 + jnp.uint32(k1)
    for g in range(5):
        for d in rots[g % 2]:
            x0 = x0 + x1
            x1 = (x1 << jnp.uint32(d)) | (x1 >> jnp.uint32(32 - d))
            x1 = x1 ^ x0
        x0 = x0 + jnp.uint32(ks[(g + 1) % 3])
        x1 = x1 + jnp.uint32((ks[(g + 2) % 3] + g + 1) & 0xFFFFFFFF)
    return x0 ^ x1


def _bits_to_unit(bits):
    fb = (bits >> jnp.uint32(9)) | jnp.uint32(0x3F800000)
    return lax.bitcast_convert_type(fb, jnp.float32) - jnp.float32(1.0)


def _kernel(pi_ref, mut_ref, lst_ref, out_ref,
            rmax_ref, ridx_ref, bmu, bls, sem_mu, sem_ls):
    j = pl.program_id(0)

    @pl.when(j == 0)
    def _init():
        rmax_ref[...] = jnp.full((B, 1), -jnp.inf, jnp.float32)
        ridx_ref[...] = jnp.zeros((B, 1), jnp.int32)

    col = lax.broadcasted_iota(jnp.int32, (B, W), 1) + j * W
    row = lax.broadcasted_iota(jnp.int32, (B, W), 0)
    p = (row * NC + col).astype(jnp.uint32)
    bits = _threefry_bits(p, _KC0, _KC1)
    fl = _bits_to_unit(bits)
    u = fl + jnp.float32(_TINY)
    g = -jnp.log(-jnp.log(u))
    score = jnp.log(pi_ref[...] + jnp.float32(1e-12)) + g
    score = jnp.where(col < NC, score, -jnp.inf)

    bmax = jnp.max(score, axis=1, keepdims=True)
    bidx = jnp.min(jnp.where(score == bmax, col, _IMAX), axis=1, keepdims=True)
    better = bmax > rmax_ref[...]
    ridx_ref[...] = jnp.where(better, bidx, ridx_ref[...])
    rmax_ref[...] = jnp.maximum(bmax, rmax_ref[...])

    @pl.when(j == NBLK - 1)
    def _finish():
        for i in range(B):
            r0 = pl.multiple_of(ridx_ref[i, 0] & ~7, 8)
            pltpu.make_async_copy(mut_ref.at[pl.ds(r0, 8)],
                                  bmu.at[i], sem_mu).start()
            pltpu.make_async_copy(lst_ref.at[pl.ds(r0, 8)],
                                  bls.at[i], sem_ls).start()

        idx = ridx_ref[...]
        rowc = lax.broadcasted_iota(jnp.int32, (B, 1), 0)
        flat = rowc * NC + idx
        bitsn = _threefry_bits(flat.astype(jnp.uint32), _KN0, _KN1)
        fln = _bits_to_unit(bitsn)
        un = jnp.maximum(jnp.float32(_LO),
                         fln * jnp.float32(_SPAN) + jnp.float32(_LO))
        eps = jnp.float32(_SQRT2) * lax.erf_inv(un)

        for i in range(B):
            pltpu.make_async_copy(mut_ref.at[pl.ds(0, 8)],
                                  bmu.at[i], sem_mu).wait()
            pltpu.make_async_copy(lst_ref.at[pl.ds(0, 8)],
                                  bls.at[i], sem_ls).wait()

        idx3 = idx.reshape(B, 1, 1)
        rio = lax.broadcasted_iota(jnp.int32, (B, 8, B), 1)
        lio = lax.broadcasted_iota(jnp.int32, (B, 8, B), 2)
        sio = lax.broadcasted_iota(jnp.int32, (B, 8, B), 0)
        sel = (lio == sio) & (rio == (idx3 & 7))
        muv = jnp.sum(jnp.where(sel, bmu[...], 0.0), axis=(1, 2),
                      keepdims=True)
        lsv = jnp.sum(jnp.where(sel, bls[...], 0.0), axis=(1, 2),
                      keepdims=True)
        sig = jnp.exp(jnp.clip(lsv, jnp.float32(-40.0), jnp.float32(40.0)))
        out_ref[...] = (muv + sig * eps.reshape(B, 1, 1)).reshape(B, 1)


_fused = pl.pallas_call(
    _kernel,
    grid=(NBLK,),
    in_specs=[pl.BlockSpec((B, W), lambda j: (0, j)),
              pl.BlockSpec(memory_space=pl.ANY),
              pl.BlockSpec(memory_space=pl.ANY)],
    out_specs=pl.BlockSpec((B, 1), lambda j: (0, 0)),
    out_shape=jax.ShapeDtypeStruct((B, 1), jnp.float32),
    scratch_shapes=[pltpu.VMEM((B, 1), jnp.float32),
                    pltpu.VMEM((B, 1), jnp.int32),
                    pltpu.VMEM((B, 8, B), jnp.float32),
                    pltpu.VMEM((B, 8, B), jnp.float32),
                    pltpu.SemaphoreType.DMA,
                    pltpu.SemaphoreType.DMA],
)


def kernel(pi, mu, log_sig):
    return _fused(pi, mu.T, log_sig.T)
